# Initial kernel scaffold; baseline (speedup 1.0000x reference)
import jax
import jax.numpy as jnp
from jax.experimental import pallas as pl


def kernel(x, W, b, wa, ba):
    raise NotImplementedError("write your pallas kernel here")


# speedup vs baseline: 1.2377x; 1.2377x over previous
import functools

import jax
import jax.numpy as jnp
from jax.experimental import pallas as pl

MASK_RATIO = 0.7


def _scores_kernel(x_ref, wt_ref, b_ref, wa_ref, ba_ref, s_ref):
    h = jnp.tanh(
        jnp.dot(x_ref[...], wt_ref[...], preferred_element_type=jnp.float32)
        + b_ref[...]
    )
    s_ref[...] = (
        jnp.dot(h, wa_ref[...], preferred_element_type=jnp.float32) + ba_ref[...]
    )


def _mask_softmax_kernel(s_ref, w_ref, *, keep_k, t_dim):
    s = s_ref[...]
    bsz = s.shape[0]
    i = jax.lax.bitcast_convert_type(s, jnp.int32)
    key = jnp.where(i >= 0, i, i ^ jnp.int32(0x7FFFFFFF))

    kk = jnp.int32(keep_k)
    n_nonneg = jnp.sum((key >= 0).astype(jnp.int32), axis=1, keepdims=True)
    sign_base = jnp.where(n_nonneg >= kk, jnp.int32(0), jnp.int32(-(2**31)))

    rv = jnp.zeros((bsz, 1), jnp.int32)
    for bit in range(30, -1, -1):
        t = rv | jnp.int32(1 << bit)
        trial = t | sign_base
        cnt = jnp.sum((key >= trial).astype(jnp.int32), axis=1, keepdims=True)
        rv = jnp.where(cnt >= kk, t, rv)
    kth = rv | sign_base

    n_gt = jnp.sum((key > kth).astype(jnp.int32), axis=1, keepdims=True)
    k_eq = kk - n_gt
    tied = key == kth
    idx = jax.lax.broadcasted_iota(jnp.int32, s.shape, 1)
    cstar = jnp.zeros((bsz, 1), jnp.int32)
    for bit in range(12, -1, -1):
        t2 = cstar | jnp.int32(1 << bit)
        cnt = jnp.sum((tied & (idx >= t2)).astype(jnp.int32), axis=1, keepdims=True)
        cstar = jnp.where(cnt >= k_eq, t2, cstar)

    hold = (key > kth) | (tied & (idx >= cstar))

    m = jnp.max(s, axis=1, keepdims=True)
    e = jnp.exp(s - m)
    denom = jnp.sum(e, axis=1, keepdims=True)
    w = e / denom
    w_ref[...] = jnp.where(hold, w, jnp.float32(0.0))


def _scale_kernel(x_ref, w_ref, o_ref):
    o_ref[...] = x_ref[...] * w_ref[...]


def kernel(x, W, b, wa, ba):
    bsz, t_dim, d = x.shape
    num_mask = int(t_dim * MASK_RATIO)
    keep_k = t_dim - num_mask
    n = bsz * t_dim
    bm = 1024
    grid = n // bm

    xf = x.reshape(n, d)
    wt = W.T
    b2 = b.reshape(1, d)
    wa2 = wa.reshape(d, 1)
    ba2 = ba.reshape(1, 1)

    scores = pl.pallas_call(
        _scores_kernel,
        grid=(grid,),
        in_specs=[
            pl.BlockSpec((bm, d), lambda i: (i, 0)),
            pl.BlockSpec((d, d), lambda i: (0, 0)),
            pl.BlockSpec((1, d), lambda i: (0, 0)),
            pl.BlockSpec((d, 1), lambda i: (0, 0)),
            pl.BlockSpec((1, 1), lambda i: (0, 0)),
        ],
        out_specs=pl.BlockSpec((bm, 1), lambda i: (i, 0)),
        out_shape=jax.ShapeDtypeStruct((n, 1), jnp.float32),
    )(xf, wt, b2, wa2, ba2)

    weights = pl.pallas_call(
        functools.partial(_mask_softmax_kernel, keep_k=keep_k, t_dim=t_dim),
        in_specs=[pl.BlockSpec((bsz, t_dim), lambda: (0, 0))],
        out_specs=pl.BlockSpec((bsz, t_dim), lambda: (0, 0)),
        out_shape=jax.ShapeDtypeStruct((bsz, t_dim), jnp.float32),
    )(scores.reshape(bsz, t_dim))

    out = pl.pallas_call(
        _scale_kernel,
        grid=(grid,),
        in_specs=[
            pl.BlockSpec((bm, d), lambda i: (i, 0)),
            pl.BlockSpec((bm, 1), lambda i: (i, 0)),
        ],
        out_specs=pl.BlockSpec((bm, d), lambda i: (i, 0)),
        out_shape=jax.ShapeDtypeStruct((n, d), jnp.float32),
    )(xf, weights.reshape(n, 1))

    return (out.reshape(bsz, t_dim, d), weights)
